# Initial kernel scaffold; baseline (speedup 1.0000x reference)
import jax
import jax.numpy as jnp
from jax.experimental import pallas as pl


def kernel(x, edge_index, W1, b1, W_ih, W_hh, b_ih, b_hh, W2, b2):
    raise NotImplementedError("write your pallas kernel here")


# speedup vs baseline: 5.4067x; 5.4067x over previous
import functools

import jax
import jax.numpy as jnp
from jax.experimental import pallas as pl
from jax.experimental.pallas import tpu as pltpu

_N = 100000
_HIST = 12
_HID = 16
_GRU_H = 32
_BLK = 2000


def _gru_body(s0_ref, s1_ref, u_ref, d_ref, w1_ref, b1_ref, wih_ref,
              whh_ref, bih_ref, bhh_ref, w2_ref, v_ref):
    d = d_ref[...]
    agg = d * (s0_ref[...] + s1_ref[...] + u_ref[...])
    w1 = w1_ref[...]
    b1 = b1_ref[...]
    wih = wih_ref[...]
    whh = whh_ref[...]
    bih = bih_ref[...]
    bhh = bhh_ref[...]
    h = jnp.zeros((_BLK, _GRU_H), jnp.float32)
    for t in range(_HIST):
        a = agg[:, t:t + 1]
        ht = jnp.maximum(a * w1 + b1, 0.0)
        gi = jnp.dot(ht, wih, preferred_element_type=jnp.float32) + bih
        gh = jnp.dot(h, whh, preferred_element_type=jnp.float32) + bhh
        r = jax.nn.sigmoid(gi[:, 0:32] + gh[:, 0:32])
        z = jax.nn.sigmoid(gi[:, 32:64] + gh[:, 32:64])
        n = jnp.tanh(gi[:, 64:96] + r * gh[:, 64:96])
        h = (1.0 - z) * n + z * h
    y = jnp.dot(h, w2_ref[...], preferred_element_type=jnp.float32)
    v_ref[...] = d * y


def _gru_dense(s0, s1, u, d, w1, b1, wih_t, whh_t, bih, bhh, w2):
    n_pad = s0.shape[0]
    grid = (n_pad // _BLK,)
    full = lambda shape: pl.BlockSpec(shape, lambda i: (0, 0))
    return pl.pallas_call(
        _gru_body,
        grid=grid,
        in_specs=[
            pl.BlockSpec((_BLK, _HIST), lambda i: (i, 0)),
            pl.BlockSpec((_BLK, _HIST), lambda i: (i, 0)),
            pl.BlockSpec((_BLK, _HIST), lambda i: (i, 0)),
            pl.BlockSpec((_BLK, 1), lambda i: (i, 0)),
            full((1, _HID)),
            full((1, _HID)),
            full((_HID, 3 * _GRU_H)),
            full((_GRU_H, 3 * _GRU_H)),
            full((1, 3 * _GRU_H)),
            full((1, 3 * _GRU_H)),
            full((_GRU_H, 1)),
        ],
        out_specs=pl.BlockSpec((_BLK, 1), lambda i: (i, 0)),
        out_shape=jax.ShapeDtypeStruct((n_pad, 1), jnp.float32),
    )(s0, s1, u, d, w1, b1, wih_t, whh_t, bih, bhh, w2)


def kernel(x, edge_index, W1, b1, W_ih, W_hh, b_ih, b_hh, W2, b2):
    n = x.shape[0]
    n_pad = 100000
    src = edge_index[0]
    dst = edge_index[1]

    deg = jnp.zeros((n,), jnp.float32).at[dst].add(1.0) + 1.0
    d = jax.lax.rsqrt(deg)

    u = d[:, None] * x
    s = jnp.zeros((n, _HIST), jnp.float32).at[dst].add(u[src])
    s0 = s
    s1 = jnp.zeros_like(s)

    v = _gru_dense(
        s0, s1, u, d[:, None], W1, b1[None, :], W_ih.T, W_hh.T,
        b_ih[None, :], b_hh[None, :], W2,
    )[:, 0]

    sy = jnp.zeros((n,), jnp.float32).at[dst].add(v[src])
    return d * (sy + v) + b2[0]


# speedup vs baseline: 66.8762x; 12.3691x over previous
import functools

import jax
import jax.numpy as jnp
from jax import lax
from jax.experimental import pallas as pl
from jax.experimental.pallas import tpu as pltpu
from jax.experimental.pallas import tpu_sc as plsc

_N = 100000
_HIST = 12
_HID = 16
_GRU_H = 32

_NC, _NS, _L = 2, 16, 16
_NW = _NC * _NS
_EPW = 50176
_E_PAD = _NW * _EPW
_CHUNK = 1024
_NCHUNK = _EPW // _CHUNK
_CROWS = _CHUNK // 128
_N_PAD = 100352
_STRIPE = _N_PAD // _NS
_BLK = 2048


@functools.lru_cache(maxsize=1)
def _sc_kernels():
    mesh = plsc.VectorSubcoreMesh(core_axis_name="c", subcore_axis_name="s")
    cparams = pltpu.CompilerParams(needs_layout_passes=False)

    deg = functools.partial(
        pl.kernel,
        out_type=jax.ShapeDtypeStruct((_NC, _N_PAD), jnp.float32),
        mesh=mesh,
        compiler_params=cparams,
        scratch_types=[
            pltpu.VMEM((_CROWS, 128), jnp.int32),
            pltpu.VMEM((128,), jnp.float32),
            pltpu.VMEM_SHARED((_N_PAD,), jnp.float32),
        ],
    )(_deg_body)
    wide = functools.partial(
        pl.kernel,
        out_type=jax.ShapeDtypeStruct((_NC, _HIST, _N_PAD), jnp.float32),
        mesh=mesh,
        compiler_params=cparams,
        scratch_types=[
            pltpu.VMEM((_N_PAD,), jnp.float32),
            pltpu.VMEM((_CHUNK,), jnp.int32),
            pltpu.VMEM((_CROWS, 128), jnp.int32),
            pltpu.VMEM((_CHUNK,), jnp.float32),
            pltpu.VMEM_SHARED((_N_PAD,), jnp.float32),
        ],
    )(_wide_body)
    scal = functools.partial(
        pl.kernel,
        out_type=jax.ShapeDtypeStruct((_NC, _N_PAD), jnp.float32),
        mesh=mesh,
        compiler_params=cparams,
        scratch_types=[
            pltpu.VMEM((_N_PAD,), jnp.float32),
            pltpu.VMEM((_CHUNK,), jnp.int32),
            pltpu.VMEM((_CROWS, 128), jnp.int32),
            pltpu.VMEM((_CHUNK,), jnp.float32),
            pltpu.VMEM_SHARED((_N_PAD,), jnp.float32),
        ],
    )(_scalar_body)
    return deg, wide, scal


def _deg_body(dst_hbm, zeros_hbm, out_hbm, idx_v, ones_v, acc_sh):
    c = lax.axis_index("c")
    s = lax.axis_index("s")
    wid = s * _NC + c
    sl = pl.ds(s * _STRIPE, _STRIPE)
    for i in range(128 // _L):
        ones_v[pl.ds(i * _L, _L)] = jnp.ones((_L,), jnp.float32)
    pltpu.sync_copy(zeros_hbm.at[sl], acc_sh.at[sl])
    plsc.subcore_barrier()
    row0 = wid * (_EPW // 128)

    def body(i, carry):
        pltpu.sync_copy(dst_hbm.at[pl.ds(row0 + i * _CROWS, _CROWS)], idx_v)
        for j in range(_CROWS):
            pltpu.sync_copy(ones_v, acc_sh.at[idx_v.at[j]], add=True)
        return carry

    lax.fori_loop(0, _NCHUNK, body, 0)
    plsc.subcore_barrier()
    pltpu.sync_copy(acc_sh.at[sl], out_hbm.at[c, sl])


def _wide_body(src_hbm, dst_hbm, ut_hbm, zeros_hbm, out_hbm,
               tab_v, sidx_v, didx_v, vals_v, acc_sh):
    c = lax.axis_index("c")
    s = lax.axis_index("s")
    wid = s * _NC + c
    sl = pl.ds(s * _STRIPE, _STRIPE)
    base = wid * _EPW
    row0 = wid * (_EPW // 128)

    def col(t, carry):
        pltpu.sync_copy(ut_hbm.at[t], tab_v)
        pltpu.sync_copy(zeros_hbm.at[sl], acc_sh.at[sl])
        plsc.subcore_barrier()

        def body(i, carry2):
            pltpu.sync_copy(src_hbm.at[pl.ds(base + i * _CHUNK, _CHUNK)],
                            sidx_v)
            pltpu.sync_copy(dst_hbm.at[pl.ds(row0 + i * _CROWS, _CROWS)],
                            didx_v)
            for k in range(_CHUNK // _L):
                idx = sidx_v[pl.ds(k * _L, _L)]
                vals_v[pl.ds(k * _L, _L)] = plsc.load_gather(tab_v, [idx])
            for j in range(_CROWS):
                pltpu.sync_copy(vals_v.at[pl.ds(j * 128, 128)],
                                acc_sh.at[didx_v.at[j]], add=True)
            return carry2

        lax.fori_loop(0, _NCHUNK, body, 0)
        plsc.subcore_barrier()
        pltpu.sync_copy(acc_sh.at[sl], out_hbm.at[c, t, sl])
        return carry

    lax.fori_loop(0, _HIST, col, 0)


def _scalar_body(src_hbm, dst_hbm, v_hbm, zeros_hbm, out_hbm,
                 vt_v, sidx_v, didx_v, vals_v, acc_sh):
    c = lax.axis_index("c")
    s = lax.axis_index("s")
    wid = s * _NC + c
    sl = pl.ds(s * _STRIPE, _STRIPE)
    pltpu.sync_copy(v_hbm, vt_v)
    pltpu.sync_copy(zeros_hbm.at[sl], acc_sh.at[sl])
    plsc.subcore_barrier()
    base = wid * _EPW
    row0 = wid * (_EPW // 128)

    def body(i, carry):
        pltpu.sync_copy(src_hbm.at[pl.ds(base + i * _CHUNK, _CHUNK)], sidx_v)
        pltpu.sync_copy(dst_hbm.at[pl.ds(row0 + i * _CROWS, _CROWS)], didx_v)
        for k in range(_CHUNK // _L):
            idx = sidx_v[pl.ds(k * _L, _L)]
            vals_v[pl.ds(k * _L, _L)] = plsc.load_gather(vt_v, [idx])
        for j in range(_CROWS):
            pltpu.sync_copy(vals_v.at[pl.ds(j * 128, 128)],
                            acc_sh.at[didx_v.at[j]], add=True)
        return carry

    lax.fori_loop(0, _NCHUNK, body, 0)
    plsc.subcore_barrier()
    pltpu.sync_copy(acc_sh.at[sl], out_hbm.at[c, sl])


def _prep_body(p_ref, xt_ref, u_ref, d_ref):
    deg = p_ref[0:1, :] + p_ref[1:2, :] + 1.0
    dd = lax.rsqrt(deg)
    d_ref[...] = dd
    u_ref[...] = dd * xt_ref[...]


def _prep(p, xt):
    grid = (_N_PAD // _BLK,)
    return pl.pallas_call(
        _prep_body,
        grid=grid,
        in_specs=[
            pl.BlockSpec((_NC, _BLK), lambda i: (0, i)),
            pl.BlockSpec((_HIST, _BLK), lambda i: (0, i)),
        ],
        out_specs=[
            pl.BlockSpec((_HIST, _BLK), lambda i: (0, i)),
            pl.BlockSpec((1, _BLK), lambda i: (0, i)),
        ],
        out_shape=[
            jax.ShapeDtypeStruct((_HIST, _N_PAD), jnp.float32),
            jax.ShapeDtypeStruct((1, _N_PAD), jnp.float32),
        ],
    )(p, xt)


def _gru_body(sp_ref, u_ref, d_ref, w1_ref, b1_ref, wih_ref,
              whh_ref, bih_ref, bhh_ref, w2_ref, v_ref):
    d = d_ref[...]
    agg = d * (sp_ref[0] + sp_ref[1] + u_ref[...])
    w1 = w1_ref[...]
    b1 = b1_ref[...]
    wih = wih_ref[...]
    whh = whh_ref[...]
    bih = bih_ref[...]
    bhh = bhh_ref[...]
    h = jnp.zeros((_GRU_H, _BLK), jnp.float32)
    for t in range(_HIST):
        a = agg[t:t + 1, :]
        ht = jnp.maximum(w1 * a + b1, 0.0)
        gi = jnp.dot(wih, ht, preferred_element_type=jnp.float32) + bih
        gh = jnp.dot(whh, h, preferred_element_type=jnp.float32) + bhh
        r = jax.nn.sigmoid(gi[0:32, :] + gh[0:32, :])
        z = jax.nn.sigmoid(gi[32:64, :] + gh[32:64, :])
        n = jnp.tanh(gi[64:96, :] + r * gh[64:96, :])
        h = (1.0 - z) * n + z * h
    y = jnp.dot(w2_ref[...], h, preferred_element_type=jnp.float32)
    v_ref[...] = d * y


def _gru_dense(sp, u, d, w1t, b1c, wih, whh, bihc, bhhc, w2t):
    grid = (_N_PAD // _BLK,)
    full = lambda shape: pl.BlockSpec(shape, lambda i: tuple(0 for _ in shape))
    return pl.pallas_call(
        _gru_body,
        grid=grid,
        in_specs=[
            pl.BlockSpec((_NC, _HIST, _BLK), lambda i: (0, 0, i)),
            pl.BlockSpec((_HIST, _BLK), lambda i: (0, i)),
            pl.BlockSpec((1, _BLK), lambda i: (0, i)),
            full((_HID, 1)),
            full((_HID, 1)),
            full((3 * _GRU_H, _HID)),
            full((3 * _GRU_H, _GRU_H)),
            full((3 * _GRU_H, 1)),
            full((3 * _GRU_H, 1)),
            full((1, _GRU_H)),
        ],
        out_specs=pl.BlockSpec((1, _BLK), lambda i: (0, i)),
        out_shape=jax.ShapeDtypeStruct((1, _N_PAD), jnp.float32),
    )(sp, u, d, w1t, b1c, wih, whh, bihc, bhhc, w2t)


def _final_body(sy_ref, v_ref, d_ref, b2_ref, o_ref):
    o_ref[...] = (d_ref[...] * (sy_ref[0:1, :] + sy_ref[1:2, :] + v_ref[...])
                  + b2_ref[...])


def _final(sy, v, d, b2):
    grid = (_N_PAD // _BLK,)
    spec = pl.BlockSpec((1, _BLK), lambda i: (0, i))
    return pl.pallas_call(
        _final_body,
        grid=grid,
        in_specs=[pl.BlockSpec((_NC, _BLK), lambda i: (0, i)), spec, spec,
                  pl.BlockSpec((1, 1), lambda i: (0, 0))],
        out_specs=spec,
        out_shape=jax.ShapeDtypeStruct((1, _N_PAD), jnp.float32),
    )(sy, v, d, b2)


def kernel(x, edge_index, W1, b1, W_ih, W_hh, b_ih, b_hh, W2, b2):
    src = edge_index[0]
    dst = edge_index[1]
    e = src.shape[0]
    fill = jnp.full((_E_PAD - e,), _N, jnp.int32)
    src_p = jnp.concatenate([src, fill])
    dst2 = jnp.concatenate([dst, fill]).reshape(_E_PAD // 128, 128)
    xt = jnp.pad(x.T, ((0, 0), (0, _N_PAD - _N)))
    z1 = jnp.zeros((_N_PAD,), jnp.float32)

    deg_k, wide_k, scal_k = _sc_kernels()
    degp = deg_k(dst2, z1)
    u, d = _prep(degp, xt)
    sp = wide_k(src_p, dst2, u, z1)
    v = _gru_dense(
        sp, u, d, W1.T, b1[:, None], W_ih, W_hh,
        b_ih[:, None], b_hh[:, None], W2.T,
    )
    syp = scal_k(src_p, dst2, v[0], z1)
    out = _final(syp, v, d, b2[None, :])
    return out[0, :_N]


# speedup vs baseline: 104.4270x; 1.5615x over previous
import functools

import jax
import jax.numpy as jnp
from jax import lax
from jax.experimental import pallas as pl
from jax.experimental.pallas import tpu as pltpu
from jax.experimental.pallas import tpu_sc as plsc

_N = 100000
_HIST = 12
_HID = 16
_GRU_H = 32

_NC, _NS, _L = 2, 16, 16
_NW = _NC * _NS
_EPW = 51200
_E_PAD = _NW * _EPW
_CHUNK = 1024
_NCHUNK = _EPW // _CHUNK
_NPAIR = _NCHUNK // 2
_CROWS = _CHUNK // 128
_N_PAD = 100352
_STRIPE = _N_PAD // _NS
_BLK = 2048


@functools.lru_cache(maxsize=1)
def _sc_kernels():
    mesh = plsc.VectorSubcoreMesh(core_axis_name="c", subcore_axis_name="s")
    cparams = pltpu.CompilerParams(needs_layout_passes=False)

    deg = functools.partial(
        pl.kernel,
        out_type=jax.ShapeDtypeStruct((_NC, _N_PAD), jnp.float32),
        mesh=mesh,
        compiler_params=cparams,
        scratch_types=[
            pltpu.VMEM((_CROWS, 128), jnp.int32),
            pltpu.VMEM((128,), jnp.float32),
            pltpu.VMEM_SHARED((_N_PAD,), jnp.float32),
        ],
    )(_deg_body)
    pipe_scratch = [
        pltpu.VMEM((_N_PAD,), jnp.float32),
        pltpu.VMEM((_CHUNK,), jnp.int32),
        pltpu.VMEM((_CHUNK,), jnp.int32),
        pltpu.VMEM((_CROWS, 128), jnp.int32),
        pltpu.VMEM((_CROWS, 128), jnp.int32),
        pltpu.VMEM((_CHUNK,), jnp.float32),
        pltpu.VMEM((_CHUNK,), jnp.float32),
        pltpu.VMEM_SHARED((_N_PAD,), jnp.float32),
        pltpu.SemaphoreType.DMA,
        pltpu.SemaphoreType.DMA,
        pltpu.SemaphoreType.DMA,
        pltpu.SemaphoreType.DMA,
    ]
    wide = functools.partial(
        pl.kernel,
        out_type=jax.ShapeDtypeStruct((_NC, _HIST, _N_PAD), jnp.float32),
        mesh=mesh,
        compiler_params=cparams,
        scratch_types=pipe_scratch,
    )(_wide_body)
    scal = functools.partial(
        pl.kernel,
        out_type=jax.ShapeDtypeStruct((_NC, _N_PAD), jnp.float32),
        mesh=mesh,
        compiler_params=cparams,
        scratch_types=pipe_scratch,
    )(_scalar_body)
    return deg, wide, scal


def _deg_body(dst_hbm, zeros_hbm, out_hbm, idx_v, ones_v, acc_sh):
    c = lax.axis_index("c")
    s = lax.axis_index("s")
    wid = s * _NC + c
    sl = pl.ds(s * _STRIPE, _STRIPE)
    for i in range(128 // _L):
        ones_v[pl.ds(i * _L, _L)] = jnp.ones((_L,), jnp.float32)
    pltpu.sync_copy(zeros_hbm.at[sl], acc_sh.at[sl])
    plsc.subcore_barrier()
    row0 = wid * (_EPW // 128)

    def body(i, carry):
        pltpu.sync_copy(dst_hbm.at[pl.ds(row0 + i * _CROWS, _CROWS)], idx_v)
        for j in range(_CROWS):
            pltpu.sync_copy(ones_v, acc_sh.at[idx_v.at[j]], add=True)
        return carry

    lax.fori_loop(0, _NCHUNK, body, 0)
    plsc.subcore_barrier()
    pltpu.sync_copy(acc_sh.at[sl], out_hbm.at[c, sl])


def _prime(src_hbm, dst_hbm, base, row0, sbuf, dbuf, isem):
    for b in range(2):
        pltpu.async_copy(src_hbm.at[pl.ds(base + b * _CHUNK, _CHUNK)],
                         sbuf[b], isem[b])
        pltpu.async_copy(dst_hbm.at[pl.ds(row0 + b * _CROWS, _CROWS)],
                         dbuf[b], isem[b])


def _edge_pass(src_hbm, dst_hbm, zeros_hbm, tab_v, acc_sh, base, row0,
               sbuf, dbuf, vbuf, isem, ssem):
    def pair(p, carry):
        for b in range(2):
            pltpu.make_async_copy(src_hbm.at[pl.ds(base, _CHUNK)],
                                  sbuf[b], isem[b]).wait()
            pltpu.make_async_copy(dst_hbm.at[pl.ds(row0, _CROWS)],
                                  dbuf[b], isem[b]).wait()
            for k in range(_CHUNK // _L):
                idx = sbuf[b][pl.ds(k * _L, _L)]
                vbuf[b][pl.ds(k * _L, _L)] = plsc.load_gather(tab_v, [idx])
            for j in range(_CROWS):
                pltpu.async_copy(vbuf[b].at[pl.ds(j * 128, 128)],
                                 acc_sh.at[dbuf[b].at[j]], ssem[b], add=True)
            pltpu.make_async_copy(zeros_hbm.at[pl.ds(0, _CHUNK)],
                                  vbuf[b], ssem[b]).wait()

            @pl.when(p < _NPAIR - 1)
            def _():
                nc = 2 * p + b + 2
                pltpu.async_copy(
                    src_hbm.at[pl.ds(base + nc * _CHUNK, _CHUNK)],
                    sbuf[b], isem[b])
                pltpu.async_copy(
                    dst_hbm.at[pl.ds(row0 + nc * _CROWS, _CROWS)],
                    dbuf[b], isem[b])
        return carry

    lax.fori_loop(0, _NPAIR, pair, 0)


def _wide_body(src_hbm, dst_hbm, ut_hbm, zeros_hbm, out_hbm,
               tab_v, sidx0, sidx1, didx0, didx1, vals0, vals1, acc_sh,
               isem0, isem1, ssem0, ssem1):
    c = lax.axis_index("c")
    s = lax.axis_index("s")
    wid = s * _NC + c
    sl = pl.ds(s * _STRIPE, _STRIPE)
    base = wid * _EPW
    row0 = wid * (_EPW // 128)
    sbuf, dbuf, vbuf = (sidx0, sidx1), (didx0, didx1), (vals0, vals1)
    isem, ssem = (isem0, isem1), (ssem0, ssem1)

    def col(t, carry):
        pltpu.sync_copy(ut_hbm.at[t], tab_v)
        pltpu.sync_copy(zeros_hbm.at[sl], acc_sh.at[sl])
        _prime(src_hbm, dst_hbm, base, row0, sbuf, dbuf, isem)
        plsc.subcore_barrier()
        _edge_pass(src_hbm, dst_hbm, zeros_hbm, tab_v, acc_sh, base, row0,
                   sbuf, dbuf, vbuf, isem, ssem)
        plsc.subcore_barrier()
        pltpu.sync_copy(acc_sh.at[sl], out_hbm.at[c, t, sl])
        return carry

    lax.fori_loop(0, _HIST, col, 0)


def _scalar_body(src_hbm, dst_hbm, v_hbm, zeros_hbm, out_hbm,
                 vt_v, sidx0, sidx1, didx0, didx1, vals0, vals1, acc_sh,
                 isem0, isem1, ssem0, ssem1):
    c = lax.axis_index("c")
    s = lax.axis_index("s")
    wid = s * _NC + c
    sl = pl.ds(s * _STRIPE, _STRIPE)
    base = wid * _EPW
    row0 = wid * (_EPW // 128)
    sbuf, dbuf, vbuf = (sidx0, sidx1), (didx0, didx1), (vals0, vals1)
    isem, ssem = (isem0, isem1), (ssem0, ssem1)
    pltpu.sync_copy(v_hbm, vt_v)
    pltpu.sync_copy(zeros_hbm.at[sl], acc_sh.at[sl])
    _prime(src_hbm, dst_hbm, base, row0, sbuf, dbuf, isem)
    plsc.subcore_barrier()
    _edge_pass(src_hbm, dst_hbm, zeros_hbm, vt_v, acc_sh, base, row0,
               sbuf, dbuf, vbuf, isem, ssem)
    plsc.subcore_barrier()
    pltpu.sync_copy(acc_sh.at[sl], out_hbm.at[c, sl])


def _prep_body(p_ref, xt_ref, u_ref, d_ref):
    deg = p_ref[0:1, :] + p_ref[1:2, :] + 1.0
    dd = lax.rsqrt(deg)
    d_ref[...] = dd
    u_ref[...] = dd * xt_ref[...]


def _prep(p, xt):
    grid = (_N_PAD // _BLK,)
    return pl.pallas_call(
        _prep_body,
        grid=grid,
        in_specs=[
            pl.BlockSpec((_NC, _BLK), lambda i: (0, i)),
            pl.BlockSpec((_HIST, _BLK), lambda i: (0, i)),
        ],
        out_specs=[
            pl.BlockSpec((_HIST, _BLK), lambda i: (0, i)),
            pl.BlockSpec((1, _BLK), lambda i: (0, i)),
        ],
        out_shape=[
            jax.ShapeDtypeStruct((_HIST, _N_PAD), jnp.float32),
            jax.ShapeDtypeStruct((1, _N_PAD), jnp.float32),
        ],
    )(p, xt)


def _gru_body(sp_ref, u_ref, d_ref, w1_ref, b1_ref, wih_ref,
              whh_ref, bih_ref, bhh_ref, w2_ref, v_ref):
    d = d_ref[...]
    agg = d * (sp_ref[0] + sp_ref[1] + u_ref[...])
    w1 = w1_ref[...]
    b1 = b1_ref[...]
    wih = wih_ref[...]
    whh = whh_ref[...]
    bih = bih_ref[...]
    bhh = bhh_ref[...]
    h = jnp.zeros((_GRU_H, _BLK), jnp.float32)
    for t in range(_HIST):
        a = agg[t:t + 1, :]
        ht = jnp.maximum(w1 * a + b1, 0.0)
        gi = jnp.dot(wih, ht, preferred_element_type=jnp.float32) + bih
        gh = jnp.dot(whh, h, preferred_element_type=jnp.float32) + bhh
        r = jax.nn.sigmoid(gi[0:32, :] + gh[0:32, :])
        z = jax.nn.sigmoid(gi[32:64, :] + gh[32:64, :])
        n = jnp.tanh(gi[64:96, :] + r * gh[64:96, :])
        h = (1.0 - z) * n + z * h
    y = jnp.dot(w2_ref[...], h, preferred_element_type=jnp.float32)
    v_ref[...] = d * y


def _gru_dense(sp, u, d, w1t, b1c, wih, whh, bihc, bhhc, w2t):
    grid = (_N_PAD // _BLK,)
    full = lambda shape: pl.BlockSpec(shape, lambda i: tuple(0 for _ in shape))
    return pl.pallas_call(
        _gru_body,
        grid=grid,
        in_specs=[
            pl.BlockSpec((_NC, _HIST, _BLK), lambda i: (0, 0, i)),
            pl.BlockSpec((_HIST, _BLK), lambda i: (0, i)),
            pl.BlockSpec((1, _BLK), lambda i: (0, i)),
            full((_HID, 1)),
            full((_HID, 1)),
            full((3 * _GRU_H, _HID)),
            full((3 * _GRU_H, _GRU_H)),
            full((3 * _GRU_H, 1)),
            full((3 * _GRU_H, 1)),
            full((1, _GRU_H)),
        ],
        out_specs=pl.BlockSpec((1, _BLK), lambda i: (0, i)),
        out_shape=jax.ShapeDtypeStruct((1, _N_PAD), jnp.float32),
    )(sp, u, d, w1t, b1c, wih, whh, bihc, bhhc, w2t)


def _final_body(sy_ref, v_ref, d_ref, b2_ref, o_ref):
    o_ref[...] = (d_ref[...] * (sy_ref[0:1, :] + sy_ref[1:2, :] + v_ref[...])
                  + b2_ref[...])


def _final(sy, v, d, b2):
    grid = (_N_PAD // _BLK,)
    spec = pl.BlockSpec((1, _BLK), lambda i: (0, i))
    return pl.pallas_call(
        _final_body,
        grid=grid,
        in_specs=[pl.BlockSpec((_NC, _BLK), lambda i: (0, i)), spec, spec,
                  pl.BlockSpec((1, 1), lambda i: (0, 0))],
        out_specs=spec,
        out_shape=jax.ShapeDtypeStruct((1, _N_PAD), jnp.float32),
    )(sy, v, d, b2)


def kernel(x, edge_index, W1, b1, W_ih, W_hh, b_ih, b_hh, W2, b2):
    src = edge_index[0]
    dst = edge_index[1]
    e = src.shape[0]
    fill = jnp.full((_E_PAD - e,), _N, jnp.int32)
    src_p = jnp.concatenate([src, fill])
    dst2 = jnp.concatenate([dst, fill]).reshape(_E_PAD // 128, 128)
    xt = jnp.pad(x.T, ((0, 0), (0, _N_PAD - _N)))
    z1 = jnp.zeros((_N_PAD,), jnp.float32)

    deg_k, wide_k, scal_k = _sc_kernels()
    degp = deg_k(dst2, z1)
    u, d = _prep(degp, xt)
    sp = wide_k(src_p, dst2, u, z1)
    v = _gru_dense(
        sp, u, d, W1.T, b1[:, None], W_ih, W_hh,
        b_ih[:, None], b_hh[:, None], W2.T,
    )
    syp = scal_k(src_p, dst2, v[0], z1)
    out = _final(syp, v, d, b2[None, :])
    return out[0, :_N]


# speedup vs baseline: 106.6013x; 1.0208x over previous
import functools

import jax
import jax.numpy as jnp
from jax import lax
from jax.experimental import pallas as pl
from jax.experimental.pallas import tpu as pltpu
from jax.experimental.pallas import tpu_sc as plsc

_N = 100000
_HIST = 12
_HID = 16
_GRU_H = 32

_NC, _NS, _L = 2, 16, 16
_NW = _NC * _NS
_EPW = 51200
_E_PAD = _NW * _EPW
_CHUNK = 1024
_NCHUNK = _EPW // _CHUNK
_NPAIR = _NCHUNK // 2
_CROWS = _CHUNK // 128
_N_PAD = 100352
_STRIPE = _N_PAD // _NS
_BLK = 2048


@functools.lru_cache(maxsize=1)
def _sc_kernels():
    mesh = plsc.VectorSubcoreMesh(core_axis_name="c", subcore_axis_name="s")
    cparams = pltpu.CompilerParams(needs_layout_passes=False)

    deg = functools.partial(
        pl.kernel,
        out_type=jax.ShapeDtypeStruct((_NC, _N_PAD), jnp.float32),
        mesh=mesh,
        compiler_params=cparams,
        scratch_types=[
            pltpu.VMEM((2 * _CROWS, 128), jnp.int32),
            pltpu.VMEM((2 * _CROWS, 128), jnp.int32),
            pltpu.VMEM((128,), jnp.float32),
            pltpu.VMEM((_CHUNK,), jnp.float32),
            pltpu.VMEM((_CHUNK,), jnp.float32),
            pltpu.VMEM_SHARED((_N_PAD,), jnp.float32),
            pltpu.SemaphoreType.DMA,
            pltpu.SemaphoreType.DMA,
            pltpu.SemaphoreType.DMA,
            pltpu.SemaphoreType.DMA,
        ],
    )(_deg_body)
    pipe_scratch = [
        pltpu.VMEM((_N_PAD,), jnp.float32),
        pltpu.VMEM((2 * _CROWS, 128), jnp.int32),
        pltpu.VMEM((2 * _CROWS, 128), jnp.int32),
        pltpu.VMEM((_CHUNK,), jnp.float32),
        pltpu.VMEM((_CHUNK,), jnp.float32),
        pltpu.VMEM_SHARED((_N_PAD,), jnp.float32),
        pltpu.SemaphoreType.DMA,
        pltpu.SemaphoreType.DMA,
        pltpu.SemaphoreType.DMA,
        pltpu.SemaphoreType.DMA,
    ]
    wide = functools.partial(
        pl.kernel,
        out_type=jax.ShapeDtypeStruct((_NC, _HIST, _N_PAD), jnp.float32),
        mesh=mesh,
        compiler_params=cparams,
        scratch_types=pipe_scratch,
    )(_wide_body)
    scal = functools.partial(
        pl.kernel,
        out_type=jax.ShapeDtypeStruct((_NC, _N_PAD), jnp.float32),
        mesh=mesh,
        compiler_params=cparams,
        scratch_types=pipe_scratch,
    )(_scalar_body)
    return deg, wide, scal


def _deg_body(cidx_hbm, zeros_hbm, out_hbm, cbuf0, cbuf1, ones_v,
              vals0, vals1, acc_sh, isem0, isem1, ssem0, ssem1):
    c = lax.axis_index("c")
    s = lax.axis_index("s")
    wid = s * _NC + c
    sl = pl.ds(s * _STRIPE, _STRIPE)
    blk0 = wid * _NCHUNK
    cbuf, vbuf = (cbuf0, cbuf1), (vals0, vals1)
    isem, ssem = (isem0, isem1), (ssem0, ssem1)
    for i in range(128 // _L):
        ones_v[pl.ds(i * _L, _L)] = jnp.ones((_L,), jnp.float32)
    pltpu.sync_copy(zeros_hbm.at[sl], acc_sh.at[sl])
    _prime(cidx_hbm, blk0, cbuf, isem)
    plsc.subcore_barrier()

    def pair(p, carry):
        for b in range(2):
            pltpu.make_async_copy(cidx_hbm.at[blk0], cbuf[b],
                                  isem[b]).wait()
            for j in range(_CROWS):
                pltpu.async_copy(ones_v, acc_sh.at[cbuf[b].at[_CROWS + j]],
                                 ssem[b], add=True)
            pltpu.make_async_copy(zeros_hbm.at[pl.ds(0, _CHUNK)],
                                  vbuf[b], ssem[b]).wait()

            @pl.when(p < _NPAIR - 1)
            def _():
                pltpu.async_copy(cidx_hbm.at[blk0 + 2 * p + b + 2],
                                 cbuf[b], isem[b])
        return carry

    lax.fori_loop(0, _NPAIR, pair, 0)
    plsc.subcore_barrier()
    pltpu.sync_copy(acc_sh.at[sl], out_hbm.at[c, sl])


def _prime(cidx_hbm, blk0, cbuf, isem):
    for b in range(2):
        pltpu.async_copy(cidx_hbm.at[blk0 + b], cbuf[b], isem[b])


def _edge_pass(cidx_hbm, zeros_hbm, tab_v, acc_sh, blk0, cbuf, vbuf,
               isem, ssem):
    def pair(p, carry):
        for b in range(2):
            pltpu.make_async_copy(cidx_hbm.at[blk0], cbuf[b],
                                  isem[b]).wait()
            for k in range(_CHUNK // _L):
                idx = cbuf[b][k // 8, pl.ds((k % 8) * _L, _L)]
                vbuf[b][pl.ds(k * _L, _L)] = plsc.load_gather(tab_v, [idx])
            for j in range(_CROWS):
                pltpu.async_copy(vbuf[b].at[pl.ds(j * 128, 128)],
                                 acc_sh.at[cbuf[b].at[_CROWS + j]],
                                 ssem[b], add=True)
            pltpu.make_async_copy(zeros_hbm.at[pl.ds(0, _CHUNK)],
                                  vbuf[b], ssem[b]).wait()

            @pl.when(p < _NPAIR - 1)
            def _():
                pltpu.async_copy(cidx_hbm.at[blk0 + 2 * p + b + 2],
                                 cbuf[b], isem[b])
        return carry

    lax.fori_loop(0, _NPAIR, pair, 0)


def _wide_body(cidx_hbm, ut_hbm, zeros_hbm, out_hbm,
               tab_v, cbuf0, cbuf1, vals0, vals1, acc_sh,
               isem0, isem1, ssem0, ssem1):
    c = lax.axis_index("c")
    s = lax.axis_index("s")
    wid = s * _NC + c
    sl = pl.ds(s * _STRIPE, _STRIPE)
    blk0 = wid * _NCHUNK
    cbuf, vbuf = (cbuf0, cbuf1), (vals0, vals1)
    isem, ssem = (isem0, isem1), (ssem0, ssem1)

    def col(t, carry):
        pltpu.sync_copy(ut_hbm.at[t], tab_v)
        pltpu.sync_copy(zeros_hbm.at[sl], acc_sh.at[sl])
        _prime(cidx_hbm, blk0, cbuf, isem)
        plsc.subcore_barrier()
        _edge_pass(cidx_hbm, zeros_hbm, tab_v, acc_sh, blk0, cbuf, vbuf,
                   isem, ssem)
        plsc.subcore_barrier()
        pltpu.sync_copy(acc_sh.at[sl], out_hbm.at[c, t, sl])
        return carry

    lax.fori_loop(0, _HIST, col, 0)


def _scalar_body(cidx_hbm, v_hbm, zeros_hbm, out_hbm,
                 vt_v, cbuf0, cbuf1, vals0, vals1, acc_sh,
                 isem0, isem1, ssem0, ssem1):
    c = lax.axis_index("c")
    s = lax.axis_index("s")
    wid = s * _NC + c
    sl = pl.ds(s * _STRIPE, _STRIPE)
    blk0 = wid * _NCHUNK
    cbuf, vbuf = (cbuf0, cbuf1), (vals0, vals1)
    isem, ssem = (isem0, isem1), (ssem0, ssem1)
    pltpu.sync_copy(v_hbm, vt_v)
    pltpu.sync_copy(zeros_hbm.at[sl], acc_sh.at[sl])
    _prime(cidx_hbm, blk0, cbuf, isem)
    plsc.subcore_barrier()
    _edge_pass(cidx_hbm, zeros_hbm, vt_v, acc_sh, blk0, cbuf, vbuf,
               isem, ssem)
    plsc.subcore_barrier()
    pltpu.sync_copy(acc_sh.at[sl], out_hbm.at[c, sl])


def _prep_body(p_ref, xt_ref, u_ref, d_ref):
    deg = p_ref[0:1, :] + p_ref[1:2, :] + 1.0
    dd = lax.rsqrt(deg)
    d_ref[...] = dd
    u_ref[...] = dd * xt_ref[...]


def _prep(p, xt):
    grid = (_N_PAD // _BLK,)
    return pl.pallas_call(
        _prep_body,
        grid=grid,
        in_specs=[
            pl.BlockSpec((_NC, _BLK), lambda i: (0, i)),
            pl.BlockSpec((_HIST, _BLK), lambda i: (0, i)),
        ],
        out_specs=[
            pl.BlockSpec((_HIST, _BLK), lambda i: (0, i)),
            pl.BlockSpec((1, _BLK), lambda i: (0, i)),
        ],
        out_shape=[
            jax.ShapeDtypeStruct((_HIST, _N_PAD), jnp.float32),
            jax.ShapeDtypeStruct((1, _N_PAD), jnp.float32),
        ],
    )(p, xt)


def _gru_body(sp_ref, u_ref, d_ref, w1_ref, b1_ref, wih_ref,
              whh_ref, bih_ref, bhh_ref, w2_ref, v_ref):
    d = d_ref[...]
    agg = d * (sp_ref[0] + sp_ref[1] + u_ref[...])
    w1 = w1_ref[...]
    b1 = b1_ref[...]
    wih = wih_ref[...]
    whh = whh_ref[...]
    bih = bih_ref[...]
    bhh = bhh_ref[...]
    h = jnp.zeros((_GRU_H, _BLK), jnp.float32)
    for t in range(_HIST):
        a = agg[t:t + 1, :]
        ht = jnp.maximum(w1 * a + b1, 0.0)
        gi = jnp.dot(wih, ht, preferred_element_type=jnp.float32) + bih
        gh = jnp.dot(whh, h, preferred_element_type=jnp.float32) + bhh
        r = jax.nn.sigmoid(gi[0:32, :] + gh[0:32, :])
        z = jax.nn.sigmoid(gi[32:64, :] + gh[32:64, :])
        n = jnp.tanh(gi[64:96, :] + r * gh[64:96, :])
        h = (1.0 - z) * n + z * h
    y = jnp.dot(w2_ref[...], h, preferred_element_type=jnp.float32)
    v_ref[...] = d * y


def _gru_dense(sp, u, d, w1t, b1c, wih, whh, bihc, bhhc, w2t):
    grid = (_N_PAD // _BLK,)
    full = lambda shape: pl.BlockSpec(shape, lambda i: tuple(0 for _ in shape))
    return pl.pallas_call(
        _gru_body,
        grid=grid,
        in_specs=[
            pl.BlockSpec((_NC, _HIST, _BLK), lambda i: (0, 0, i)),
            pl.BlockSpec((_HIST, _BLK), lambda i: (0, i)),
            pl.BlockSpec((1, _BLK), lambda i: (0, i)),
            full((_HID, 1)),
            full((_HID, 1)),
            full((3 * _GRU_H, _HID)),
            full((3 * _GRU_H, _GRU_H)),
            full((3 * _GRU_H, 1)),
            full((3 * _GRU_H, 1)),
            full((1, _GRU_H)),
        ],
        out_specs=pl.BlockSpec((1, _BLK), lambda i: (0, i)),
        out_shape=jax.ShapeDtypeStruct((1, _N_PAD), jnp.float32),
    )(sp, u, d, w1t, b1c, wih, whh, bihc, bhhc, w2t)


def _final_body(sy_ref, v_ref, d_ref, b2_ref, o_ref):
    o_ref[...] = (d_ref[...] * (sy_ref[0:1, :] + sy_ref[1:2, :] + v_ref[...])
                  + b2_ref[...])


def _final(sy, v, d, b2):
    grid = (_N_PAD // _BLK,)
    spec = pl.BlockSpec((1, _BLK), lambda i: (0, i))
    return pl.pallas_call(
        _final_body,
        grid=grid,
        in_specs=[pl.BlockSpec((_NC, _BLK), lambda i: (0, i)), spec, spec,
                  pl.BlockSpec((1, 1), lambda i: (0, 0))],
        out_specs=spec,
        out_shape=jax.ShapeDtypeStruct((1, _N_PAD), jnp.float32),
    )(sy, v, d, b2)


def kernel(x, edge_index, W1, b1, W_ih, W_hh, b_ih, b_hh, W2, b2):
    src = edge_index[0]
    dst = edge_index[1]
    e = src.shape[0]
    fill = jnp.full((_E_PAD - e,), _N, jnp.int32)
    src4 = jnp.concatenate([src, fill]).reshape(_NW, _NCHUNK, _CROWS, 128)
    dst4 = jnp.concatenate([dst, fill]).reshape(_NW, _NCHUNK, _CROWS, 128)
    cidx = jnp.concatenate([src4, dst4], axis=2)
    cidx = cidx.reshape(_NW * _NCHUNK, 2 * _CROWS, 128)
    xt = jnp.pad(x.T, ((0, 0), (0, _N_PAD - _N)))
    z1 = jnp.zeros((_N_PAD,), jnp.float32)

    deg_k, wide_k, scal_k = _sc_kernels()
    degp = deg_k(cidx, z1)
    u, d = _prep(degp, xt)
    sp = wide_k(cidx, u, z1)
    v = _gru_dense(
        sp, u, d, W1.T, b1[:, None], W_ih, W_hh,
        b_ih[:, None], b_hh[:, None], W2.T,
    )
    syp = scal_k(cidx, v[0], z1)
    out = _final(syp, v, d, b2[None, :])
    return out[0, :_N]


# speedup vs baseline: 112.5186x; 1.0555x over previous
import functools

import jax
import jax.numpy as jnp
from jax import lax
from jax.experimental import pallas as pl
from jax.experimental.pallas import tpu as pltpu
from jax.experimental.pallas import tpu_sc as plsc

_N = 100000
_HIST = 12
_HID = 16
_GRU_H = 32

_NC, _NS, _L = 2, 16, 16
_NW = _NC * _NS
_EPW = 51200
_E_PAD = _NW * _EPW
_CHUNK = 1024
_NCHUNK = _EPW // _CHUNK
_NPAIR = _NCHUNK // 2
_CROWS = _CHUNK // 128
_N_PAD = 100352
_STRIPE = _N_PAD // _NS
_BLK = 2048


@functools.lru_cache(maxsize=1)
def _sc_kernels():
    mesh = plsc.VectorSubcoreMesh(core_axis_name="c", subcore_axis_name="s")
    cparams = pltpu.CompilerParams(needs_layout_passes=False)

    deg = functools.partial(
        pl.kernel,
        out_type=jax.ShapeDtypeStruct((_NC, _N_PAD), jnp.float32),
        mesh=mesh,
        compiler_params=cparams,
        scratch_types=[
            pltpu.VMEM((2 * _CROWS, 128), jnp.int32),
            pltpu.VMEM((2 * _CROWS, 128), jnp.int32),
            pltpu.VMEM((128,), jnp.float32),
            pltpu.VMEM((_CHUNK,), jnp.float32),
            pltpu.VMEM((_CHUNK,), jnp.float32),
            pltpu.VMEM_SHARED((_N_PAD,), jnp.float32),
            pltpu.SemaphoreType.DMA,
            pltpu.SemaphoreType.DMA,
            pltpu.SemaphoreType.DMA,
            pltpu.SemaphoreType.DMA,
        ],
    )(_deg_body)
    pipe_scratch = [
        pltpu.VMEM((_N_PAD,), jnp.float32),
        pltpu.VMEM((2 * _CROWS, 128), jnp.int32),
        pltpu.VMEM((2 * _CROWS, 128), jnp.int32),
        pltpu.VMEM((_CHUNK,), jnp.float32),
        pltpu.VMEM((_CHUNK,), jnp.float32),
        pltpu.VMEM_SHARED((_N_PAD,), jnp.float32),
        pltpu.SemaphoreType.DMA,
        pltpu.SemaphoreType.DMA,
        pltpu.SemaphoreType.DMA,
        pltpu.SemaphoreType.DMA,
    ]
    wide = functools.partial(
        pl.kernel,
        out_type=jax.ShapeDtypeStruct((_NC, _HIST, _N_PAD), jnp.float32),
        mesh=mesh,
        compiler_params=cparams,
        scratch_types=pipe_scratch,
    )(_wide_body)
    scal = functools.partial(
        pl.kernel,
        out_type=jax.ShapeDtypeStruct((_NC, _N_PAD), jnp.float32),
        mesh=mesh,
        compiler_params=cparams,
        scratch_types=pipe_scratch,
    )(_scalar_body)
    return deg, wide, scal


def _deg_body(cidx_hbm, zeros_hbm, out_hbm, cbuf0, cbuf1, ones_v,
              vals0, vals1, acc_sh, isem0, isem1, ssem0, ssem1):
    c = lax.axis_index("c")
    s = lax.axis_index("s")
    wid = s * _NC + c
    sl = pl.ds(s * _STRIPE, _STRIPE)
    blk0 = wid * _NCHUNK
    cbuf, vbuf = (cbuf0, cbuf1), (vals0, vals1)
    isem, ssem = (isem0, isem1), (ssem0, ssem1)
    for i in range(128 // _L):
        ones_v[pl.ds(i * _L, _L)] = jnp.ones((_L,), jnp.float32)
    pltpu.sync_copy(zeros_hbm.at[sl], acc_sh.at[sl])
    _prime(cidx_hbm, blk0, cbuf, isem)
    plsc.subcore_barrier()

    def pair(p, carry):
        for b in range(2):
            pltpu.make_async_copy(cidx_hbm.at[blk0], cbuf[b],
                                  isem[b]).wait()
            for j in range(_CROWS):
                pltpu.async_copy(ones_v, acc_sh.at[cbuf[b].at[_CROWS + j]],
                                 ssem[b], add=True)
            pltpu.make_async_copy(zeros_hbm.at[pl.ds(0, _CHUNK)],
                                  vbuf[b], ssem[b]).wait()

            @pl.when(p < _NPAIR - 1)
            def _():
                pltpu.async_copy(cidx_hbm.at[blk0 + 2 * p + b + 2],
                                 cbuf[b], isem[b])
        return carry

    lax.fori_loop(0, _NPAIR, pair, 0)
    plsc.subcore_barrier()
    pltpu.sync_copy(acc_sh.at[sl], out_hbm.at[c, sl])


def _prime(cidx_hbm, blk0, cbuf, isem):
    for b in range(2):
        pltpu.async_copy(cidx_hbm.at[blk0 + b], cbuf[b], isem[b])


def _edge_pass(cidx_hbm, zeros_hbm, tab_v, acc_sh, blk0, cbuf, vbuf,
               isem, ssem):
    def pair(p, carry):
        for b in range(2):
            pltpu.make_async_copy(cidx_hbm.at[blk0], cbuf[b],
                                  isem[b]).wait()
            cb, vb = cbuf[b], vbuf[b]

            @plsc.parallel_loop(0, _CHUNK, _L, unroll=8)
            def _gather(i):
                idx = cb[i // 128, pl.ds(lax.rem(i, 128), _L)]
                vb[pl.ds(i, _L)] = plsc.load_gather(tab_v, [idx])
            for j in range(_CROWS):
                pltpu.async_copy(vbuf[b].at[pl.ds(j * 128, 128)],
                                 acc_sh.at[cbuf[b].at[_CROWS + j]],
                                 ssem[b], add=True)
            pltpu.make_async_copy(zeros_hbm.at[pl.ds(0, _CHUNK)],
                                  vbuf[b], ssem[b]).wait()

            @pl.when(p < _NPAIR - 1)
            def _():
                pltpu.async_copy(cidx_hbm.at[blk0 + 2 * p + b + 2],
                                 cbuf[b], isem[b])
        return carry

    lax.fori_loop(0, _NPAIR, pair, 0)


def _wide_body(cidx_hbm, ut_hbm, zeros_hbm, out_hbm,
               tab_v, cbuf0, cbuf1, vals0, vals1, acc_sh,
               isem0, isem1, ssem0, ssem1):
    c = lax.axis_index("c")
    s = lax.axis_index("s")
    wid = s * _NC + c
    sl = pl.ds(s * _STRIPE, _STRIPE)
    blk0 = wid * _NCHUNK
    cbuf, vbuf = (cbuf0, cbuf1), (vals0, vals1)
    isem, ssem = (isem0, isem1), (ssem0, ssem1)

    def col(t, carry):
        pltpu.sync_copy(ut_hbm.at[t], tab_v)
        pltpu.sync_copy(zeros_hbm.at[sl], acc_sh.at[sl])
        _prime(cidx_hbm, blk0, cbuf, isem)
        plsc.subcore_barrier()
        _edge_pass(cidx_hbm, zeros_hbm, tab_v, acc_sh, blk0, cbuf, vbuf,
                   isem, ssem)
        plsc.subcore_barrier()
        pltpu.sync_copy(acc_sh.at[sl], out_hbm.at[c, t, sl])
        return carry

    lax.fori_loop(0, _HIST, col, 0)


def _scalar_body(cidx_hbm, v_hbm, zeros_hbm, out_hbm,
                 vt_v, cbuf0, cbuf1, vals0, vals1, acc_sh,
                 isem0, isem1, ssem0, ssem1):
    c = lax.axis_index("c")
    s = lax.axis_index("s")
    wid = s * _NC + c
    sl = pl.ds(s * _STRIPE, _STRIPE)
    blk0 = wid * _NCHUNK
    cbuf, vbuf = (cbuf0, cbuf1), (vals0, vals1)
    isem, ssem = (isem0, isem1), (ssem0, ssem1)
    pltpu.sync_copy(v_hbm, vt_v)
    pltpu.sync_copy(zeros_hbm.at[sl], acc_sh.at[sl])
    _prime(cidx_hbm, blk0, cbuf, isem)
    plsc.subcore_barrier()
    _edge_pass(cidx_hbm, zeros_hbm, vt_v, acc_sh, blk0, cbuf, vbuf,
               isem, ssem)
    plsc.subcore_barrier()
    pltpu.sync_copy(acc_sh.at[sl], out_hbm.at[c, sl])


def _prep_body(p_ref, xt_ref, u_ref, d_ref):
    deg = p_ref[0:1, :] + p_ref[1:2, :] + 1.0
    dd = lax.rsqrt(deg)
    d_ref[...] = dd
    u_ref[...] = dd * xt_ref[...]


def _prep(p, xt):
    grid = (_N_PAD // _BLK,)
    return pl.pallas_call(
        _prep_body,
        grid=grid,
        in_specs=[
            pl.BlockSpec((_NC, _BLK), lambda i: (0, i)),
            pl.BlockSpec((_HIST, _BLK), lambda i: (0, i)),
        ],
        out_specs=[
            pl.BlockSpec((_HIST, _BLK), lambda i: (0, i)),
            pl.BlockSpec((1, _BLK), lambda i: (0, i)),
        ],
        out_shape=[
            jax.ShapeDtypeStruct((_HIST, _N_PAD), jnp.float32),
            jax.ShapeDtypeStruct((1, _N_PAD), jnp.float32),
        ],
    )(p, xt)


def _gru_body(sp_ref, u_ref, d_ref, w1_ref, b1_ref, wih_ref,
              whh_ref, bih_ref, bhh_ref, w2_ref, v_ref):
    d = d_ref[...]
    agg = d * (sp_ref[0] + sp_ref[1] + u_ref[...])
    w1 = w1_ref[...]
    b1 = b1_ref[...]
    wih = wih_ref[...]
    whh = whh_ref[...]
    bih = bih_ref[...]
    bhh = bhh_ref[...]
    h = jnp.zeros((_GRU_H, _BLK), jnp.float32)
    for t in range(_HIST):
        a = agg[t:t + 1, :]
        ht = jnp.maximum(w1 * a + b1, 0.0)
        gi = jnp.dot(wih, ht, preferred_element_type=jnp.float32) + bih
        gh = jnp.dot(whh, h, preferred_element_type=jnp.float32) + bhh
        r = jax.nn.sigmoid(gi[0:32, :] + gh[0:32, :])
        z = jax.nn.sigmoid(gi[32:64, :] + gh[32:64, :])
        n = jnp.tanh(gi[64:96, :] + r * gh[64:96, :])
        h = (1.0 - z) * n + z * h
    y = jnp.dot(w2_ref[...], h, preferred_element_type=jnp.float32)
    v_ref[...] = d * y


def _gru_dense(sp, u, d, w1t, b1c, wih, whh, bihc, bhhc, w2t):
    grid = (_N_PAD // _BLK,)
    full = lambda shape: pl.BlockSpec(shape, lambda i: tuple(0 for _ in shape))
    return pl.pallas_call(
        _gru_body,
        grid=grid,
        in_specs=[
            pl.BlockSpec((_NC, _HIST, _BLK), lambda i: (0, 0, i)),
            pl.BlockSpec((_HIST, _BLK), lambda i: (0, i)),
            pl.BlockSpec((1, _BLK), lambda i: (0, i)),
            full((_HID, 1)),
            full((_HID, 1)),
            full((3 * _GRU_H, _HID)),
            full((3 * _GRU_H, _GRU_H)),
            full((3 * _GRU_H, 1)),
            full((3 * _GRU_H, 1)),
            full((1, _GRU_H)),
        ],
        out_specs=pl.BlockSpec((1, _BLK), lambda i: (0, i)),
        out_shape=jax.ShapeDtypeStruct((1, _N_PAD), jnp.float32),
    )(sp, u, d, w1t, b1c, wih, whh, bihc, bhhc, w2t)


def _final_body(sy_ref, v_ref, d_ref, b2_ref, o_ref):
    o_ref[...] = (d_ref[...] * (sy_ref[0:1, :] + sy_ref[1:2, :] + v_ref[...])
                  + b2_ref[...])


def _final(sy, v, d, b2):
    grid = (_N_PAD // _BLK,)
    spec = pl.BlockSpec((1, _BLK), lambda i: (0, i))
    return pl.pallas_call(
        _final_body,
        grid=grid,
        in_specs=[pl.BlockSpec((_NC, _BLK), lambda i: (0, i)), spec, spec,
                  pl.BlockSpec((1, 1), lambda i: (0, 0))],
        out_specs=spec,
        out_shape=jax.ShapeDtypeStruct((1, _N_PAD), jnp.float32),
    )(sy, v, d, b2)


def kernel(x, edge_index, W1, b1, W_ih, W_hh, b_ih, b_hh, W2, b2):
    src = edge_index[0]
    dst = edge_index[1]
    e = src.shape[0]
    fill = jnp.full((_E_PAD - e,), _N, jnp.int32)
    src4 = jnp.concatenate([src, fill]).reshape(_NW, _NCHUNK, _CROWS, 128)
    dst4 = jnp.concatenate([dst, fill]).reshape(_NW, _NCHUNK, _CROWS, 128)
    cidx = jnp.concatenate([src4, dst4], axis=2)
    cidx = cidx.reshape(_NW * _NCHUNK, 2 * _CROWS, 128)
    xt = jnp.pad(x.T, ((0, 0), (0, _N_PAD - _N)))
    z1 = jnp.zeros((_N_PAD,), jnp.float32)

    deg_k, wide_k, scal_k = _sc_kernels()
    degp = deg_k(cidx, z1)
    u, d = _prep(degp, xt)
    sp = wide_k(cidx, u, z1)
    v = _gru_dense(
        sp, u, d, W1.T, b1[:, None], W_ih, W_hh,
        b_ih[:, None], b_hh[:, None], W2.T,
    )
    syp = scal_k(cidx, v[0], z1)
    out = _final(syp, v, d, b2[None, :])
    return out[0, :_N]


# speedup vs baseline: 113.0016x; 1.0043x over previous
import functools

import jax
import jax.numpy as jnp
from jax import lax
from jax.experimental import pallas as pl
from jax.experimental.pallas import tpu as pltpu
from jax.experimental.pallas import tpu_sc as plsc

_N = 100000
_HIST = 12
_HID = 16
_GRU_H = 32

_NC, _NS, _L = 2, 16, 16
_NW = _NC * _NS
_EPW = 51200
_E_PAD = _NW * _EPW
_CHUNK = 1024
_NCHUNK = _EPW // _CHUNK
_NPAIR = _NCHUNK // 2
_CROWS = _CHUNK // 128
_N_PAD = 100352
_STRIPE = _N_PAD // _NS
_BLK = 2048


@functools.lru_cache(maxsize=1)
def _sc_kernels():
    mesh = plsc.VectorSubcoreMesh(core_axis_name="c", subcore_axis_name="s")
    cparams = pltpu.CompilerParams(needs_layout_passes=False)

    deg = functools.partial(
        pl.kernel,
        out_type=jax.ShapeDtypeStruct((_NC, _N_PAD), jnp.float32),
        mesh=mesh,
        compiler_params=cparams,
        scratch_types=[
            pltpu.VMEM((2 * _CROWS, 128), jnp.int32),
            pltpu.VMEM((2 * _CROWS, 128), jnp.int32),
            pltpu.VMEM((128,), jnp.float32),
            pltpu.VMEM((_CHUNK,), jnp.float32),
            pltpu.VMEM((_CHUNK,), jnp.float32),
            pltpu.VMEM_SHARED((_N_PAD,), jnp.float32),
            pltpu.SemaphoreType.DMA,
            pltpu.SemaphoreType.DMA,
            pltpu.SemaphoreType.DMA,
            pltpu.SemaphoreType.DMA,
        ],
    )(_deg_body)
    pipe_scratch = [
        pltpu.VMEM((_N_PAD,), jnp.float32),
        pltpu.VMEM((2 * _CROWS, 128), jnp.int32),
        pltpu.VMEM((2 * _CROWS, 128), jnp.int32),
        pltpu.VMEM((_CHUNK,), jnp.float32),
        pltpu.VMEM((_CHUNK,), jnp.float32),
        pltpu.VMEM_SHARED((_N_PAD,), jnp.float32),
        pltpu.SemaphoreType.DMA,
        pltpu.SemaphoreType.DMA,
        pltpu.SemaphoreType.DMA,
        pltpu.SemaphoreType.DMA,
        pltpu.SemaphoreType.DMA,
    ]
    wide = functools.partial(
        pl.kernel,
        out_type=jax.ShapeDtypeStruct((_NC, _HIST, _N_PAD), jnp.float32),
        mesh=mesh,
        compiler_params=cparams,
        scratch_types=pipe_scratch,
    )(_wide_body)
    scal = functools.partial(
        pl.kernel,
        out_type=jax.ShapeDtypeStruct((_NC, _N_PAD), jnp.float32),
        mesh=mesh,
        compiler_params=cparams,
        scratch_types=pipe_scratch,
    )(_scalar_body)
    return deg, wide, scal


def _deg_body(cidx_hbm, zeros_hbm, out_hbm, cbuf0, cbuf1, ones_v,
              vals0, vals1, acc_sh, isem0, isem1, ssem0, ssem1):
    c = lax.axis_index("c")
    s = lax.axis_index("s")
    wid = s * _NC + c
    sl = pl.ds(s * _STRIPE, _STRIPE)
    blk0 = wid * _NCHUNK
    cbuf, vbuf = (cbuf0, cbuf1), (vals0, vals1)
    isem, ssem = (isem0, isem1), (ssem0, ssem1)
    for i in range(128 // _L):
        ones_v[pl.ds(i * _L, _L)] = jnp.ones((_L,), jnp.float32)
    pltpu.sync_copy(zeros_hbm.at[sl], acc_sh.at[sl])
    _prime(cidx_hbm, blk0, cbuf, isem)
    plsc.subcore_barrier()

    def pair(p, carry):
        for b in range(2):
            pltpu.make_async_copy(cidx_hbm.at[blk0], cbuf[b],
                                  isem[b]).wait()
            for j in range(_CROWS):
                pltpu.async_copy(ones_v, acc_sh.at[cbuf[b].at[_CROWS + j]],
                                 ssem[b], add=True)
            pltpu.make_async_copy(zeros_hbm.at[pl.ds(0, _CHUNK)],
                                  vbuf[b], ssem[b]).wait()

            @pl.when(p < _NPAIR - 1)
            def _():
                pltpu.async_copy(cidx_hbm.at[blk0 + 2 * p + b + 2],
                                 cbuf[b], isem[b])
        return carry

    lax.fori_loop(0, _NPAIR, pair, 0)
    plsc.subcore_barrier()
    pltpu.sync_copy(acc_sh.at[sl], out_hbm.at[c, sl])


def _prime(cidx_hbm, blk0, cbuf, isem):
    for b in range(2):
        pltpu.async_copy(cidx_hbm.at[blk0 + b], cbuf[b], isem[b])


def _edge_pass(cidx_hbm, zeros_hbm, tab_v, acc_sh, blk0, cbuf, vbuf,
               isem, ssem):
    def pair(p, carry):
        for b in range(2):
            pltpu.make_async_copy(cidx_hbm.at[blk0], cbuf[b],
                                  isem[b]).wait()
            cb, vb = cbuf[b], vbuf[b]

            @plsc.parallel_loop(0, _CHUNK, _L, unroll=8)
            def _gather(i):
                idx = cb[i // 128, pl.ds(lax.rem(i, 128), _L)]
                vb[pl.ds(i, _L)] = plsc.load_gather(tab_v, [idx])
            for j in range(_CROWS):
                pltpu.async_copy(vbuf[b].at[pl.ds(j * 128, 128)],
                                 acc_sh.at[cbuf[b].at[_CROWS + j]],
                                 ssem[b], add=True)
            pltpu.make_async_copy(zeros_hbm.at[pl.ds(0, _CHUNK)],
                                  vbuf[b], ssem[b]).wait()

            @pl.when(p < _NPAIR - 1)
            def _():
                pltpu.async_copy(cidx_hbm.at[blk0 + 2 * p + b + 2],
                                 cbuf[b], isem[b])
        return carry

    lax.fori_loop(0, _NPAIR, pair, 0)


def _wide_body(cidx_hbm, ut_hbm, zeros_hbm, out_hbm,
               tab_v, cbuf0, cbuf1, vals0, vals1, acc_sh,
               isem0, isem1, ssem0, ssem1, tsem):
    c = lax.axis_index("c")
    s = lax.axis_index("s")
    wid = s * _NC + c
    sl = pl.ds(s * _STRIPE, _STRIPE)
    blk0 = wid * _NCHUNK
    cbuf, vbuf = (cbuf0, cbuf1), (vals0, vals1)
    isem, ssem = (isem0, isem1), (ssem0, ssem1)
    pltpu.async_copy(ut_hbm.at[0], tab_v, tsem)

    def col(t, carry):
        pltpu.sync_copy(zeros_hbm.at[sl], acc_sh.at[sl])
        _prime(cidx_hbm, blk0, cbuf, isem)
        pltpu.make_async_copy(ut_hbm.at[t], tab_v, tsem).wait()
        plsc.subcore_barrier()
        _edge_pass(cidx_hbm, zeros_hbm, tab_v, acc_sh, blk0, cbuf, vbuf,
                   isem, ssem)
        plsc.subcore_barrier()

        @pl.when(t < _HIST - 1)
        def _():
            pltpu.async_copy(ut_hbm.at[t + 1], tab_v, tsem)

        pltpu.sync_copy(acc_sh.at[sl], out_hbm.at[c, t, sl])
        return carry

    lax.fori_loop(0, _HIST, col, 0)


def _scalar_body(cidx_hbm, v_hbm, zeros_hbm, out_hbm,
                 vt_v, cbuf0, cbuf1, vals0, vals1, acc_sh,
                 isem0, isem1, ssem0, ssem1, tsem):
    c = lax.axis_index("c")
    s = lax.axis_index("s")
    wid = s * _NC + c
    sl = pl.ds(s * _STRIPE, _STRIPE)
    blk0 = wid * _NCHUNK
    cbuf, vbuf = (cbuf0, cbuf1), (vals0, vals1)
    isem, ssem = (isem0, isem1), (ssem0, ssem1)
    pltpu.async_copy(v_hbm, vt_v, tsem)
    pltpu.sync_copy(zeros_hbm.at[sl], acc_sh.at[sl])
    _prime(cidx_hbm, blk0, cbuf, isem)
    pltpu.make_async_copy(v_hbm, vt_v, tsem).wait()
    plsc.subcore_barrier()
    _edge_pass(cidx_hbm, zeros_hbm, vt_v, acc_sh, blk0, cbuf, vbuf,
               isem, ssem)
    plsc.subcore_barrier()
    pltpu.sync_copy(acc_sh.at[sl], out_hbm.at[c, sl])


def _prep_body(p_ref, xt_ref, u_ref, d_ref):
    deg = p_ref[0:1, :] + p_ref[1:2, :] + 1.0
    dd = lax.rsqrt(deg)
    d_ref[...] = dd
    u_ref[...] = dd * xt_ref[...]


def _prep(p, xt):
    grid = (_N_PAD // _BLK,)
    return pl.pallas_call(
        _prep_body,
        grid=grid,
        in_specs=[
            pl.BlockSpec((_NC, _BLK), lambda i: (0, i)),
            pl.BlockSpec((_HIST, _BLK), lambda i: (0, i)),
        ],
        out_specs=[
            pl.BlockSpec((_HIST, _BLK), lambda i: (0, i)),
            pl.BlockSpec((1, _BLK), lambda i: (0, i)),
        ],
        out_shape=[
            jax.ShapeDtypeStruct((_HIST, _N_PAD), jnp.float32),
            jax.ShapeDtypeStruct((1, _N_PAD), jnp.float32),
        ],
    )(p, xt)


def _gru_body(sp_ref, u_ref, d_ref, w1_ref, b1_ref, wih_ref,
              whh_ref, bih_ref, bhh_ref, w2_ref, v_ref):
    d = d_ref[...]
    agg = d * (sp_ref[0] + sp_ref[1] + u_ref[...])
    w1 = w1_ref[...]
    b1 = b1_ref[...]
    wih = wih_ref[...]
    whh = whh_ref[...]
    bih = bih_ref[...]
    bhh = bhh_ref[...]
    h = jnp.zeros((_GRU_H, _BLK), jnp.float32)
    for t in range(_HIST):
        a = agg[t:t + 1, :]
        ht = jnp.maximum(w1 * a + b1, 0.0)
        gi = jnp.dot(wih, ht, preferred_element_type=jnp.float32) + bih
        gh = jnp.dot(whh, h, preferred_element_type=jnp.float32) + bhh
        r = jax.nn.sigmoid(gi[0:32, :] + gh[0:32, :])
        z = jax.nn.sigmoid(gi[32:64, :] + gh[32:64, :])
        n = jnp.tanh(gi[64:96, :] + r * gh[64:96, :])
        h = (1.0 - z) * n + z * h
    y = jnp.dot(w2_ref[...], h, preferred_element_type=jnp.float32)
    v_ref[...] = d * y


def _gru_dense(sp, u, d, w1t, b1c, wih, whh, bihc, bhhc, w2t):
    grid = (_N_PAD // _BLK,)
    full = lambda shape: pl.BlockSpec(shape, lambda i: tuple(0 for _ in shape))
    return pl.pallas_call(
        _gru_body,
        grid=grid,
        in_specs=[
            pl.BlockSpec((_NC, _HIST, _BLK), lambda i: (0, 0, i)),
            pl.BlockSpec((_HIST, _BLK), lambda i: (0, i)),
            pl.BlockSpec((1, _BLK), lambda i: (0, i)),
            full((_HID, 1)),
            full((_HID, 1)),
            full((3 * _GRU_H, _HID)),
            full((3 * _GRU_H, _GRU_H)),
            full((3 * _GRU_H, 1)),
            full((3 * _GRU_H, 1)),
            full((1, _GRU_H)),
        ],
        out_specs=pl.BlockSpec((1, _BLK), lambda i: (0, i)),
        out_shape=jax.ShapeDtypeStruct((1, _N_PAD), jnp.float32),
    )(sp, u, d, w1t, b1c, wih, whh, bihc, bhhc, w2t)


def _final_body(sy_ref, v_ref, d_ref, b2_ref, o_ref):
    o_ref[...] = (d_ref[...] * (sy_ref[0:1, :] + sy_ref[1:2, :] + v_ref[...])
                  + b2_ref[...])


def _final(sy, v, d, b2):
    grid = (_N_PAD // _BLK,)
    spec = pl.BlockSpec((1, _BLK), lambda i: (0, i))
    return pl.pallas_call(
        _final_body,
        grid=grid,
        in_specs=[pl.BlockSpec((_NC, _BLK), lambda i: (0, i)), spec, spec,
                  pl.BlockSpec((1, 1), lambda i: (0, 0))],
        out_specs=spec,
        out_shape=jax.ShapeDtypeStruct((1, _N_PAD), jnp.float32),
    )(sy, v, d, b2)


def kernel(x, edge_index, W1, b1, W_ih, W_hh, b_ih, b_hh, W2, b2):
    src = edge_index[0]
    dst = edge_index[1]
    e = src.shape[0]
    fill = jnp.full((_E_PAD - e,), _N, jnp.int32)
    src4 = jnp.concatenate([src, fill]).reshape(_NW, _NCHUNK, _CROWS, 128)
    dst4 = jnp.concatenate([dst, fill]).reshape(_NW, _NCHUNK, _CROWS, 128)
    cidx = jnp.concatenate([src4, dst4], axis=2)
    cidx = cidx.reshape(_NW * _NCHUNK, 2 * _CROWS, 128)
    xt = jnp.pad(x.T, ((0, 0), (0, _N_PAD - _N)))
    z1 = jnp.zeros((_N_PAD,), jnp.float32)

    deg_k, wide_k, scal_k = _sc_kernels()
    degp = deg_k(cidx, z1)
    u, d = _prep(degp, xt)
    sp = wide_k(cidx, u, z1)
    v = _gru_dense(
        sp, u, d, W1.T, b1[:, None], W_ih, W_hh,
        b_ih[:, None], b_hh[:, None], W2.T,
    )
    syp = scal_k(cidx, v[0], z1)
    out = _final(syp, v, d, b2[None, :])
    return out[0, :_N]
